# Initial kernel scaffold; baseline (speedup 1.0000x reference)
import jax
import jax.numpy as jnp
from jax.experimental import pallas as pl


def kernel(x_field, x, table):
    raise NotImplementedError("write your pallas kernel here")


# speedup vs baseline: 24.2769x; 24.2769x over previous
import functools

import jax
import jax.numpy as jnp
from jax import lax
from jax.experimental import pallas as pl
from jax.experimental.pallas import tpu as pltpu
from jax.experimental.pallas import tpu_sc as plsc

BATCH = 4096
NNZ = 26
NUM_FIELDS = 26
FIELD_DIM = 38461
EMBED_DIM = 16
OUT_FIELDS = NUM_FIELDS - 1

NW = 32
BW = BATCH // NW
TW = BW * NNZ
CHUNK = 128
NCHUNK = TW // CHUNK
ACC_ROWS = BW * OUT_FIELDS

DIV26_MAGIC = 5042
DIV26_SHIFT = 17


def _sc_body(xf_hbm, x_hbm, table_hbm, out_hbm,
             xf_v, x_v, idx_v, dst_v, rows_v, acc_sh, zbuf_v,
             zsem, gsem):
    c_id = lax.axis_index("c")
    s_id = lax.axis_index("s")
    wid = s_id * 2 + c_id
    abase = s_id * ACC_ROWS

    def zfill(i, carry):
        zbuf_v[i, :] = jnp.zeros((16,), jnp.float32)
        return carry
    lax.fori_loop(0, BW, zfill, 0)

    def zfire(i, carry):
        pltpu.async_copy(zbuf_v, acc_sh.at[pl.ds(abase + i * BW, BW)], zsem)
        return carry
    lax.fori_loop(0, OUT_FIELDS, zfire, 0)

    pltpu.sync_copy(xf_hbm.at[pl.ds(wid * TW, TW)], xf_v)
    pltpu.sync_copy(x_hbm.at[pl.ds(wid * TW, TW)], x_v)

    def cbody(c, carry):
        for k in range(CHUNK // 16):
            t0 = c * CHUNK + k * 16
            tv = t0 + lax.iota(jnp.int32, 16)
            xv = x_v[pl.ds(t0, 16)]
            fv = xf_v[pl.ds(t0, 16)]
            valid = fv >= 1
            g = jnp.where(valid, xv + fv * FIELD_DIM, 0)
            lb = (tv * DIV26_MAGIC) >> DIV26_SHIFT
            d = abase + lb * OUT_FIELDS + jnp.where(valid, fv - 1, 0)
            idx_v[c, pl.ds(k * 16, 16)] = g
            dst_v[c, pl.ds(k * 16, 16)] = d
        return carry
    lax.fori_loop(0, NCHUNK, cbody, 0)

    def zdrain(i, carry):
        pltpu.make_async_copy(
            zbuf_v, acc_sh.at[pl.ds(abase + i * BW, BW)], zsem).wait()
        return carry
    lax.fori_loop(0, OUT_FIELDS, zdrain, 0)

    pltpu.async_copy(table_hbm.at[idx_v.at[0]], rows_v.at[0], gsem.at[0])
    pltpu.async_copy(table_hbm.at[idx_v.at[1]], rows_v.at[1], gsem.at[1])

    def outer(h, carry):
        for b in range(2):
            c = h * 2 + b
            pltpu.make_async_copy(
                table_hbm.at[idx_v.at[c]], rows_v.at[b], gsem.at[b]).wait()
            pltpu.sync_copy(rows_v.at[b], acc_sh.at[dst_v.at[c]], add=True)

            @pl.when(c + 2 < NCHUNK)
            def _():
                pltpu.async_copy(
                    table_hbm.at[idx_v.at[c + 2]], rows_v.at[b], gsem.at[b])
        return carry
    lax.fori_loop(0, NCHUNK // 2, outer, 0)

    pltpu.sync_copy(acc_sh.at[pl.ds(abase, ACC_ROWS)],
                    out_hbm.at[pl.ds(wid * ACC_ROWS, ACC_ROWS)])


@jax.jit
def _features_embedding(x_field, x, table):
    mesh = plsc.VectorSubcoreMesh(core_axis_name="c", subcore_axis_name="s")
    run = pl.kernel(
        _sc_body,
        out_type=jax.ShapeDtypeStruct((BATCH * OUT_FIELDS, EMBED_DIM),
                                      jnp.float32),
        mesh=mesh,
        compiler_params=pltpu.CompilerParams(use_tc_tiling_on_sc=False),
        scratch_types=[
            pltpu.VMEM((TW,), jnp.int32),
            pltpu.VMEM((TW,), jnp.int32),
            pltpu.VMEM((NCHUNK, CHUNK), jnp.int32),
            pltpu.VMEM((NCHUNK, CHUNK), jnp.int32),
            pltpu.VMEM((2, CHUNK, EMBED_DIM), jnp.float32),
            pltpu.VMEM_SHARED((16 * ACC_ROWS, EMBED_DIM), jnp.float32),
            pltpu.VMEM((BW, EMBED_DIM), jnp.float32),
            pltpu.SemaphoreType.DMA,
            pltpu.SemaphoreType.DMA((2,)),
        ],
    )
    out = run(x_field.reshape(-1).astype(jnp.int32),
              x.reshape(-1).astype(jnp.int32),
              table)
    return out.reshape(BATCH, OUT_FIELDS, EMBED_DIM)


def kernel(x_field, x, table):
    return _features_embedding(x_field, x, table)
